# baseline (speedup 1.0000x reference)
import functools

import jax
import jax.numpy as jnp
from jax import lax
from jax.experimental import pallas as pl
from jax.experimental.pallas import tpu as pltpu
from jax.experimental.pallas import tpu_sc as plsc

B = 16384
DIM = 64
PAIRW = 2 * DIM
NUM_CORES = 2
NUM_SUBCORES = 16
NW = NUM_CORES * NUM_SUBCORES
RPW = B // NW
CHUNK = 256
NCHUNK = RPW // CHUNK
LANES = 16
GROUPS = CHUNK // LANES


def _body(u_hbm, i_hbm, j_hbm, uf_hbm, if_hbm, bias_hbm, out_hbm,
          ui, ii, ji, pu, pi, pj, uv, iv, jv, bi, bj, ov, sem):
    wid = lax.axis_index("s") * NUM_CORES + lax.axis_index("c")
    base = wid * RPW

    pltpu.sync_copy(u_hbm.at[pl.ds(base, RPW)], ui)
    pltpu.sync_copy(i_hbm.at[pl.ds(base, RPW)], ii)
    pltpu.sync_copy(j_hbm.at[pl.ds(base, RPW)], ji)

    def halve(src, dst, t):
        v = src[pl.ds(t * LANES, LANES)]
        dst[pl.ds(t * LANES, LANES)] = lax.shift_right_logical(v, 1)

    def halve_all(t, carry):
        halve(ui, pu, t)
        halve(ii, pi, t)
        halve(ji, pj, t)
        return carry

    lax.fori_loop(0, RPW // LANES, halve_all, 0)

    lanes = lax.iota(jnp.int32, LANES)
    one = jnp.full((LANES,), 1, jnp.int32)

    def chunk_pass(c, carry):
        cb = c * CHUNK
        g1 = pltpu.async_copy(uf_hbm.at[pu.at[pl.ds(cb, CHUNK)]], uv, sem)
        g2 = pltpu.async_copy(if_hbm.at[pi.at[pl.ds(cb, CHUNK)]], iv, sem)
        g3 = pltpu.async_copy(if_hbm.at[pj.at[pl.ds(cb, CHUNK)]], jv, sem)
        g4 = pltpu.async_copy(bias_hbm.at[ii.at[pl.ds(cb, CHUNK)]], bi, sem)
        g5 = pltpu.async_copy(bias_hbm.at[ji.at[pl.ds(cb, CHUNK)]], bj, sem)
        g1.wait()
        g2.wait()
        g3.wait()
        g4.wait()
        g5.wait()

        def group(g, carry2):
            rb = g * LANES
            row_idx = lanes + rb
            hu = lax.shift_left((ui[pl.ds(cb + rb, LANES)] & one), 6)
            hi = lax.shift_left((ii[pl.ds(cb + rb, LANES)] & one), 6)
            hj = lax.shift_left((ji[pl.ds(cb + rb, LANES)] & one), 6)
            acc = bi[pl.ds(rb, LANES)] - bj[pl.ds(rb, LANES)]

            def dstep(d, a):
                dv = jnp.full((LANES,), d, jnp.int32)
                uu = plsc.load_gather(uv, [row_idx, hu + dv])
                xi = plsc.load_gather(iv, [row_idx, hi + dv])
                xj = plsc.load_gather(jv, [row_idx, hj + dv])
                return a + uu * (xi - xj)

            acc = lax.fori_loop(0, DIM, dstep, acc, unroll=8)
            ov[pl.ds(cb + rb, LANES)] = acc
            return carry2

        lax.fori_loop(0, GROUPS, group, carry)
        return carry

    lax.fori_loop(0, NCHUNK, chunk_pass, 0)
    pltpu.sync_copy(ov, out_hbm.at[pl.ds(base, RPW)])


@functools.partial(jax.jit, static_argnames=())
def kernel(u, i, j, user_factors, item_factors, item_biases):
    mesh = plsc.VectorSubcoreMesh(core_axis_name="c", subcore_axis_name="s")
    k = functools.partial(
        pl.kernel,
        mesh=mesh,
        compiler_params=pltpu.CompilerParams(needs_layout_passes=False),
        out_type=jax.ShapeDtypeStruct((B,), jnp.float32),
        scratch_types=[
            pltpu.VMEM((RPW,), jnp.int32),
            pltpu.VMEM((RPW,), jnp.int32),
            pltpu.VMEM((RPW,), jnp.int32),
            pltpu.VMEM((RPW,), jnp.int32),
            pltpu.VMEM((RPW,), jnp.int32),
            pltpu.VMEM((RPW,), jnp.int32),
            pltpu.VMEM((CHUNK, PAIRW), jnp.float32),
            pltpu.VMEM((CHUNK, PAIRW), jnp.float32),
            pltpu.VMEM((CHUNK, PAIRW), jnp.float32),
            pltpu.VMEM((CHUNK,), jnp.float32),
            pltpu.VMEM((CHUNK,), jnp.float32),
            pltpu.VMEM((RPW,), jnp.float32),
            pltpu.SemaphoreType.DMA,
        ],
    )(_body)
    uf2 = user_factors.reshape(-1, PAIRW)
    if2 = item_factors.reshape(-1, PAIRW)
    bias_flat = item_biases.reshape(-1)
    return k(u, i, j, uf2, if2, bias_flat)


# speedup vs baseline: 1.0045x; 1.0045x over previous
import functools

import jax
import jax.numpy as jnp
from jax import lax
from jax.experimental import pallas as pl
from jax.experimental.pallas import tpu as pltpu
from jax.experimental.pallas import tpu_sc as plsc

B = 16384
DIM = 64
NUM_ROWS = 1000000
NUM_CORES = 2
NUM_SUBCORES = 16
NW = NUM_CORES * NUM_SUBCORES
RPW = B // NW
LANES = 16
GROUPS = RPW // LANES


def _body(u_hbm, i_hbm, j_hbm, uf_hbm, if_hbm, bias_hbm, out_hbm,
          ui, ii, ji, uv, iv, jv, bi, bj, ov, sem):
    wid = lax.axis_index("s") * NUM_CORES + lax.axis_index("c")
    base = wid * RPW

    pltpu.sync_copy(u_hbm.at[pl.ds(base, RPW)], ui)
    pltpu.sync_copy(i_hbm.at[pl.ds(base, RPW)], ii)
    pltpu.sync_copy(j_hbm.at[pl.ds(base, RPW)], ji)

    c4 = pltpu.async_copy(bias_hbm.at[ii], bi, sem)
    c5 = pltpu.async_copy(bias_hbm.at[ji], bj, sem)

    dimv = jnp.full((LANES,), DIM, jnp.int32)

    def fire(g, carry):
        gb = g * LANES
        uo = ui[pl.ds(gb, LANES)] * dimv
        io = ii[pl.ds(gb, LANES)] * dimv
        jo = ji[pl.ds(gb, LANES)] * dimv
        for l in range(LANES):
            dst = pl.ds((gb + l) * DIM, DIM)
            uol = pl.multiple_of(uo[l], DIM)
            iol = pl.multiple_of(io[l], DIM)
            jol = pl.multiple_of(jo[l], DIM)
            pltpu.async_copy(uf_hbm.at[pl.ds(uol, DIM)], uv.at[dst], sem)
            pltpu.async_copy(if_hbm.at[pl.ds(iol, DIM)], iv.at[dst], sem)
            pltpu.async_copy(if_hbm.at[pl.ds(jol, DIM)], jv.at[dst], sem)
        return carry

    lax.fori_loop(0, GROUPS, fire, 0)

    pltpu.make_async_copy(uf_hbm.at[pl.ds(0, RPW * DIM)], uv, sem).wait()
    pltpu.make_async_copy(uf_hbm.at[pl.ds(0, RPW * DIM)], iv, sem).wait()
    pltpu.make_async_copy(uf_hbm.at[pl.ds(0, RPW * DIM)], jv, sem).wait()
    c4.wait()
    c5.wait()

    lanes = lax.iota(jnp.int32, LANES)

    def group(g, carry):
        rb = g * LANES
        flat_base = lax.mul(lanes + rb, jnp.full((LANES,), DIM, jnp.int32))
        acc = bi[pl.ds(rb, LANES)] - bj[pl.ds(rb, LANES)]

        def dstep(d, a):
            fidx = flat_base + jnp.full((LANES,), d, jnp.int32)
            uu = plsc.load_gather(uv, [fidx])
            xi = plsc.load_gather(iv, [fidx])
            xj = plsc.load_gather(jv, [fidx])
            return a + uu * (xi - xj)

        acc = lax.fori_loop(0, DIM, dstep, acc, unroll=8)
        ov[pl.ds(rb, LANES)] = acc
        return carry

    lax.fori_loop(0, GROUPS, group, 0)
    pltpu.sync_copy(ov, out_hbm.at[pl.ds(base, RPW)])


@functools.partial(jax.jit, static_argnames=())
def kernel(u, i, j, user_factors, item_factors, item_biases):
    mesh = plsc.VectorSubcoreMesh(core_axis_name="c", subcore_axis_name="s")
    k = functools.partial(
        pl.kernel,
        mesh=mesh,
        compiler_params=pltpu.CompilerParams(needs_layout_passes=False),
        out_type=jax.ShapeDtypeStruct((B,), jnp.float32),
        scratch_types=[
            pltpu.VMEM((RPW,), jnp.int32),
            pltpu.VMEM((RPW,), jnp.int32),
            pltpu.VMEM((RPW,), jnp.int32),
            pltpu.VMEM((RPW * DIM,), jnp.float32),
            pltpu.VMEM((RPW * DIM,), jnp.float32),
            pltpu.VMEM((RPW * DIM,), jnp.float32),
            pltpu.VMEM((RPW,), jnp.float32),
            pltpu.VMEM((RPW,), jnp.float32),
            pltpu.VMEM((RPW,), jnp.float32),
            pltpu.SemaphoreType.DMA,
        ],
    )(_body)
    uf1 = user_factors.reshape(-1)
    if1 = item_factors.reshape(-1)
    bias_flat = item_biases.reshape(-1)
    return k(u, i, j, uf1, if1, bias_flat)
